# baseline (speedup 1.0000x reference)
import jax
import jax.numpy as jnp
from jax.experimental import pallas as pl
from jax.experimental.pallas import tpu as pltpu

_LANE = 128
_BLOCK = 40960


def kernel(input_ids, scores, allowed_ids):
    del input_ids
    batch, vocab = scores.shape
    nids = allowed_ids.shape[0]
    num_blocks = pl.cdiv(vocab, _BLOCK)

    def body(*refs):
        ids_ref = refs[0]
        score_refs = refs[1:1 + nids]
        out_ref = refs[1 + nids]
        i = pl.program_id(0)
        base = i * _BLOCK
        out_ref[...] = jnp.full((batch, _BLOCK), -jnp.inf, out_ref.dtype)
        coliota = jax.lax.broadcasted_iota(jnp.int32, (batch, _BLOCK), 1)
        laneiota = jax.lax.broadcasted_iota(jnp.int32, (batch, _LANE), 1)
        for j in range(nids):
            pos = ids_ref[j] - base

            @pl.when((pos >= 0) & (pos < _BLOCK))
            def _place(j=j, pos=pos):
                c = ids_ref[j] % _LANE
                col = jnp.sum(
                    jnp.where(laneiota == c, score_refs[j][...], 0.0),
                    axis=1, keepdims=True)
                out_ref[...] = jnp.where(coliota == pos, col, out_ref[...])

    in_specs = [
        pl.BlockSpec((batch, _LANE), (lambda i, ids, j=j: (0, ids[j] // _LANE)))
        for j in range(nids)
    ]
    out = pl.pallas_call(
        body,
        grid_spec=pltpu.PrefetchScalarGridSpec(
            num_scalar_prefetch=1,
            grid=(num_blocks,),
            in_specs=in_specs,
            out_specs=pl.BlockSpec((batch, _BLOCK), lambda i, ids: (0, i)),
        ),
        out_shape=jax.ShapeDtypeStruct((batch, vocab), scores.dtype),
    )(allowed_ids, *([scores] * nids))
    return out


# speedup vs baseline: 7.5568x; 7.5568x over previous
import jax
import jax.numpy as jnp
from jax import lax
from jax.experimental import pallas as pl
from jax.experimental.pallas import tpu as pltpu

_LANE = 128
_BLOCK = 32768
_NBUF = 4


def kernel(input_ids, scores, allowed_ids):
    del input_ids
    batch, vocab = scores.shape
    nids = allowed_ids.shape[0]
    nblk = vocab // _BLOCK
    ngroups = (nblk + _NBUF - 1) // _NBUF

    in_specs = [
        pl.BlockSpec((batch, _LANE), (lambda i, ids, j=j: (0, ids[j] // _LANE)))
        for j in range(nids)
    ]
    laneiota = None

    def ring_body(*refs):
        ids_ref = refs[0]
        score_refs = refs[1:1 + nids]
        out_ref = refs[1 + nids]
        scratch = refs[2 + nids:2 + nids + _NBUF]
        sems = refs[2 + nids + _NBUF]
        laneiota = jax.lax.broadcasted_iota(jnp.int32, (batch, _LANE), 1)
        neg_inf_tile = jnp.full((batch, _LANE), -jnp.inf, jnp.float32)

        def patch(r, b, restore):
            base = b * _BLOCK
            for j in range(nids):
                pos = ids_ref[j] - base

                @pl.when((pos >= 0) & (pos < _BLOCK))
                def _p(j=j, pos=pos):
                    t = pl.multiple_of((pos // _LANE) * _LANE, _LANE)
                    if restore:
                        scratch[r][:, pl.ds(t, _LANE)] = neg_inf_tile
                    else:
                        c = ids_ref[j] % _LANE
                        col = jnp.sum(
                            jnp.where(laneiota == c, score_refs[j][...], 0.0),
                            axis=1, keepdims=True)
                        cur = scratch[r][:, pl.ds(t, _LANE)]
                        scratch[r][:, pl.ds(t, _LANE)] = jnp.where(
                            laneiota + t == pos, col, cur)

        def group(g, carry):
            for r in range(_NBUF):
                b = g * _NBUF + r

                @pl.when(b < nblk)
                def _do(b=b, r=r):
                    @pl.when(b < _NBUF)
                    def _init(r=r):
                        scratch[r][...] = jnp.full(
                            (batch, _BLOCK), -jnp.inf, jnp.float32)

                    @pl.when(b >= _NBUF)
                    def _reuse(b=b, r=r):
                        pltpu.make_async_copy(
                            scratch[r],
                            out_ref.at[:, pl.ds((b - _NBUF) * _BLOCK, _BLOCK)],
                            sems.at[r]).wait()
                        patch(r, b - _NBUF, restore=True)

                    patch(r, b, restore=False)
                    pltpu.make_async_copy(
                        scratch[r],
                        out_ref.at[:, pl.ds(b * _BLOCK, _BLOCK)],
                        sems.at[r]).start()
            return carry

        lax.fori_loop(0, ngroups, group, 0)
        for r in range(_NBUF):
            last_b = nblk - 1 - (nblk - 1 - r) % _NBUF
            pltpu.make_async_copy(
                scratch[r],
                out_ref.at[:, pl.ds(last_b * _BLOCK, _BLOCK)],
                sems.at[r]).wait()

    main = pl.pallas_call(
        ring_body,
        grid_spec=pltpu.PrefetchScalarGridSpec(
            num_scalar_prefetch=1,
            grid=(1,),
            in_specs=in_specs,
            out_specs=pl.BlockSpec(memory_space=pltpu.MemorySpace.HBM),
            scratch_shapes=(
                [pltpu.VMEM((batch, _BLOCK), jnp.float32) for _ in range(_NBUF)]
                + [pltpu.SemaphoreType.DMA((_NBUF,))]
            ),
        ),
        out_shape=jax.ShapeDtypeStruct((batch, vocab), scores.dtype),
    )(allowed_ids, *([scores] * nids))

    def tail_body(*refs):
        ids_ref = refs[0]
        score_refs = refs[1:1 + nids]
        out_ref = refs[2 + nids]
        base = nblk * _BLOCK
        out_ref[...] = jnp.full((batch, _BLOCK), -jnp.inf, out_ref.dtype)
        coliota = jax.lax.broadcasted_iota(jnp.int32, (batch, _BLOCK), 1)
        laneiota = jax.lax.broadcasted_iota(jnp.int32, (batch, _LANE), 1)
        for j in range(nids):
            pos = ids_ref[j] - base

            @pl.when((pos >= 0) & (pos < _BLOCK))
            def _place(j=j, pos=pos):
                c = ids_ref[j] % _LANE
                col = jnp.sum(
                    jnp.where(laneiota == c, score_refs[j][...], 0.0),
                    axis=1, keepdims=True)
                out_ref[...] = jnp.where(coliota == pos, col, out_ref[...])

    out = pl.pallas_call(
        tail_body,
        grid_spec=pltpu.PrefetchScalarGridSpec(
            num_scalar_prefetch=1,
            grid=(1,),
            in_specs=in_specs + [pl.BlockSpec(memory_space=pltpu.MemorySpace.HBM)],
            out_specs=pl.BlockSpec((batch, _BLOCK), lambda i, ids: (0, nblk)),
        ),
        out_shape=jax.ShapeDtypeStruct((batch, vocab), scores.dtype),
        input_output_aliases={nids + 1: 0},
    )(allowed_ids, *([scores] * nids), main)
    return out


# speedup vs baseline: 8.0028x; 1.0590x over previous
import jax
import jax.numpy as jnp
from jax.experimental import pallas as pl
from jax.experimental.pallas import tpu as pltpu

_LANE = 128
_BLOCK = 32768


def kernel(input_ids, scores, allowed_ids):
    del input_ids
    batch, vocab = scores.shape
    nids = allowed_ids.shape[0]
    num_blocks = pl.cdiv(vocab, _BLOCK)

    def body(*refs):
        ids_ref = refs[0]
        score_refs = refs[1:1 + nids]
        out_ref = refs[1 + nids]
        i = pl.program_id(0)
        base = i * _BLOCK
        out_ref[...] = jnp.full((batch, _BLOCK), -jnp.inf, out_ref.dtype)
        coliota = jax.lax.broadcasted_iota(jnp.int32, (batch, _BLOCK), 1)
        laneiota = jax.lax.broadcasted_iota(jnp.int32, (batch, _LANE), 1)
        for j in range(nids):
            pos = ids_ref[j] - base

            @pl.when((pos >= 0) & (pos < _BLOCK))
            def _place(j=j, pos=pos):
                c = ids_ref[j] % _LANE
                col = jnp.sum(
                    jnp.where(laneiota == c, score_refs[j][...], 0.0),
                    axis=1, keepdims=True)
                out_ref[...] = jnp.where(coliota == pos, col, out_ref[...])

    in_specs = [
        pl.BlockSpec((batch, _LANE), (lambda i, ids, j=j: (0, ids[j] // _LANE)))
        for j in range(nids)
    ]
    out = pl.pallas_call(
        body,
        grid_spec=pltpu.PrefetchScalarGridSpec(
            num_scalar_prefetch=1,
            grid=(num_blocks,),
            in_specs=in_specs,
            out_specs=pl.BlockSpec((batch, _BLOCK), lambda i, ids: (0, i)),
        ),
        out_shape=jax.ShapeDtypeStruct((batch, vocab), scores.dtype),
    )(allowed_ids, *([scores] * nids))
    return out
